# baseline (speedup 1.0000x reference)
import functools

import jax
import jax.numpy as jnp
from jax import lax
from jax.experimental import pallas as pl
from jax.experimental.pallas import tpu as pltpu
from jax.experimental.pallas import tpu_sc as plsc

N = 10000
E = 320000
D = 128
HD = D // 2
NC = 2
NS = 16
L = 16
EPS = E // NS
CHUNK = 128
STEPS = 160
EPP = STEPS * CHUNK
NBUF = 4
GROUPS = STEPS // NBUF
NPAD = 10240
RPS = NPAD // NS


def _seg_sum_call(feats2, srcx, dst4, zrow, zcnt, ones_pat, with_counts):
  mesh = plsc.VectorSubcoreMesh(
      core_axis_name="c", subcore_axis_name="s", num_cores=NC, num_subcores=NS)

  out_type = [jax.ShapeDtypeStruct((NC * NPAD, HD), jnp.float32)]
  scratch = {
      "sidx": pltpu.VMEM((STEPS, CHUNK), jnp.int32),
      "didx": pltpu.VMEM((STEPS, CHUNK), jnp.int32),
      "acc": pltpu.VMEM_SHARED((NPAD, HD), jnp.float32),
  }
  for b in range(NBUF):
    scratch[f"rows{b}"] = pltpu.VMEM((CHUNK, HD), jnp.float32)
    scratch[f"gsem{b}"] = pltpu.SemaphoreType.DMA
    scratch[f"ssem{b}"] = pltpu.SemaphoreType.DMA
    if with_counts:
      scratch[f"csem{b}"] = pltpu.SemaphoreType.DMA
  if with_counts:
    out_type.append(jax.ShapeDtypeStruct((NPAD, L), jnp.float32))
    out_type.append(jax.ShapeDtypeStruct((NPAD, L), jnp.float32))
    scratch.update({
        "ones_v": pltpu.VMEM((CHUNK, L), jnp.float32),
        "cacc": pltpu.VMEM_SHARED((NPAD, L), jnp.float32),
    })

  def body(feats_h, src_h, dst_h, zrow_h, zcnt_h, ones_h, *outs, **sc):
    out_h = outs[0]
    core = lax.axis_index("c")
    sub = lax.axis_index("s")
    row0 = sub * RPS
    sidx, didx = sc["sidx"], sc["didx"]
    rows = [sc[f"rows{b}"] for b in range(NBUF)]
    gsem = [sc[f"gsem{b}"] for b in range(NBUF)]
    ssem = [sc[f"ssem{b}"] for b in range(NBUF)]
    acc = sc["acc"]

    pltpu.sync_copy(src_h.at[pl.ds((core * NS + sub) * STEPS, STEPS)], sidx)
    pltpu.sync_copy(dst_h.at[pl.ds(sub * STEPS, STEPS)], didx)

    pltpu.sync_copy(zrow_h, acc.at[pl.ds(row0, RPS)])
    if with_counts:
      pltpu.sync_copy(ones_h, sc["ones_v"])
      pltpu.sync_copy(zcnt_h, sc["cacc"].at[pl.ds(row0, RPS)])
    plsc.subcore_barrier()

    for b in range(NBUF):
      pltpu.async_copy(feats_h.at[sidx.at[b]], rows[b], gsem[b])

    def group(j, carry):
      k0 = j * NBUF
      handles = []
      for b in range(NBUF):
        k = k0 + b
        pltpu.make_async_copy(feats_h.at[sidx.at[k]], rows[b], gsem[b]).wait()
        handles.append(
            pltpu.async_copy(rows[b], acc.at[didx.at[k]], sem=ssem[b],
                             add=True))
        if with_counts:

          @pl.when(core == (b % 2))
          def _():
            pltpu.async_copy(sc["ones_v"], sc["cacc"].at[didx.at[k]],
                             sem=sc[f"csem{b}"], add=True)

      for b in range(NBUF):
        k = k0 + b
        handles[b].wait()
        if with_counts:

          @pl.when(core == (b % 2))
          def _():
            pltpu.make_async_copy(
                sc["ones_v"], sc["cacc"].at[didx.at[k]], sc[f"csem{b}"]).wait()

        @pl.when(k + NBUF < STEPS)
        def _():
          pltpu.async_copy(feats_h.at[sidx.at[k + NBUF]], rows[b], gsem[b])

      return carry

    lax.fori_loop(0, GROUPS, group, 0)
    plsc.subcore_barrier()

    pltpu.sync_copy(acc.at[pl.ds(row0, RPS)],
                    out_h.at[pl.ds(core * NPAD + row0, RPS)])
    if with_counts:

      @pl.when(core == 0)
      def _():
        pltpu.sync_copy(sc["cacc"].at[pl.ds(row0, RPS)],
                        outs[1].at[pl.ds(row0, RPS)])

      @pl.when(core == 1)
      def _():
        pltpu.sync_copy(sc["cacc"].at[pl.ds(row0, RPS)],
                        outs[2].at[pl.ds(row0, RPS)])

  fn = pl.kernel(
      body, out_type=out_type, mesh=mesh, scratch_types=scratch,
      compiler_params=pltpu.CompilerParams(use_tc_tiling_on_sc=False))
  return fn(feats2, srcx, dst4, zrow, zcnt, ones_pat)


def _dense_body(layer1, p_l, p_r, c0, c1, xr, wlt, bl, wrt, scale, beta, o):
  cnt = jnp.maximum(c0[:, 0:1] + c1[:, 0:1], 1.0)
  agg_l = p_l[...] / cnt
  agg_r = p_r[...] / cnt
  h = (jnp.dot(agg_l, wlt[0:HD, :], preferred_element_type=jnp.float32)
       + jnp.dot(agg_r, wlt[HD:D, :], preferred_element_type=jnp.float32)
       + bl[...]
       + jnp.dot(xr[...], wrt[...], preferred_element_type=jnp.float32))
  nrm = jnp.sqrt(jnp.sum(h * h, axis=1, keepdims=True))
  h = h / jnp.maximum(nrm, 1e-12)
  if layer1:
    h = h * scale[...] + beta[...]
    h = jnp.maximum(h, 0.0)
  o[...] = h


def _dense_call(layer1, P, C0, C1, xin, wlt, bl, wrt, scale, beta):
  R = 640
  NB = NPAD // R
  specs = [
      pl.BlockSpec((R, HD), lambda i: (i, 0)),
      pl.BlockSpec((R, HD), lambda i: (i + NB, 0)),
      pl.BlockSpec((R, L), lambda i: (i, 0)),
      pl.BlockSpec((R, L), lambda i: (i, 0)),
      pl.BlockSpec((R, D), lambda i: (i, 0)),
      pl.BlockSpec((D, D), lambda i: (0, 0)),
      pl.BlockSpec((1, D), lambda i: (0, 0)),
      pl.BlockSpec((D, D), lambda i: (0, 0)),
      pl.BlockSpec((1, D), lambda i: (0, 0)),
      pl.BlockSpec((1, D), lambda i: (0, 0)),
  ]
  return pl.pallas_call(
      functools.partial(_dense_body, layer1),
      grid=(NB,),
      in_specs=specs,
      out_specs=pl.BlockSpec((R, D), lambda i: (i, 0)),
      out_shape=jax.ShapeDtypeStruct((N, D), jnp.float32),
  )(P, P, C0, C1, xin, wlt, bl, wrt, scale, beta)


def kernel(x, edge_index, Wl1, bl1, Wr1, bn_gamma, bn_beta, Wl2, bl2, Wr2):
  src = jnp.pad(edge_index[0].astype(jnp.int32).reshape(NS, EPS),
                ((0, 0), (0, EPP - EPS))).reshape(-1)
  srcx = jnp.concatenate([2 * src, 2 * src + 1]).reshape(
      NC * NS * STEPS, CHUNK)
  dst4 = jnp.pad(edge_index[1].astype(jnp.int32).reshape(NS, EPS),
                 ((0, 0), (0, EPP - EPS)),
                 constant_values=N + 100).reshape(NS * STEPS, CHUNK)
  zrow = jnp.zeros((RPS, HD), jnp.float32)
  zcnt = jnp.zeros((RPS, L), jnp.float32)
  ones_pat = jnp.zeros((CHUNK, L), jnp.float32).at[:, 0].set(1.0)

  P1, C0, C1 = _seg_sum_call(x.reshape(2 * N, HD), srcx, dst4, zrow, zcnt,
                             ones_pat, with_counts=True)
  scale1 = (bn_gamma / jnp.sqrt(1.0 + 1e-5)).reshape(1, D)
  h1 = _dense_call(True, P1, C0, C1, x, Wl1.T, bl1.reshape(1, D), Wr1.T,
                   scale1, bn_beta.reshape(1, D))

  (P2,) = _seg_sum_call(h1.reshape(2 * N, HD), srcx, dst4, zrow, zcnt,
                        ones_pat, with_counts=False)
  zb = jnp.zeros((1, D), jnp.float32)
  out = _dense_call(False, P2, C0, C1, h1, Wl2.T, bl2.reshape(1, D), Wr2.T,
                    zb, zb)
  return out


# speedup vs baseline: 2.7194x; 2.7194x over previous
import functools

import jax
import jax.numpy as jnp
from jax import lax
from jax.experimental import pallas as pl
from jax.experimental.pallas import tpu as pltpu
from jax.experimental.pallas import tpu_sc as plsc

N = 10000
E = 320000
D = 128
HD = D // 2
NC = 2
NS = 16
L = 16
EPS = E // NS
CHUNK = 80
STEPS = EPS // CHUNK
NBUF = 5
GROUPS = STEPS // NBUF
NPAD = 10240
RPS = NPAD // NS


def _seg_sum_call(feats2, srcx, dst4, zrow, zcnt, ones_pat, with_counts):
  mesh = plsc.VectorSubcoreMesh(
      core_axis_name="c", subcore_axis_name="s", num_cores=NC, num_subcores=NS)

  out_type = [jax.ShapeDtypeStruct((NC * NPAD, HD), jnp.float32)]
  scratch = {
      "sidx": pltpu.VMEM((STEPS, CHUNK), jnp.int32),
      "didx": pltpu.VMEM((STEPS, CHUNK), jnp.int32),
      "acc": pltpu.VMEM_SHARED((NPAD, HD), jnp.float32),
  }
  for b in range(NBUF):
    scratch[f"rows{b}"] = pltpu.VMEM((CHUNK, HD), jnp.float32)
    scratch[f"gsem{b}"] = pltpu.SemaphoreType.DMA
    scratch[f"ssem{b}"] = pltpu.SemaphoreType.DMA
    if with_counts:
      scratch[f"csem{b}"] = pltpu.SemaphoreType.DMA
  if with_counts:
    out_type.append(jax.ShapeDtypeStruct((NPAD, L), jnp.float32))
    out_type.append(jax.ShapeDtypeStruct((NPAD, L), jnp.float32))
    scratch.update({
        "ones_v": pltpu.VMEM((CHUNK, L), jnp.float32),
        "cacc": pltpu.VMEM_SHARED((NPAD, L), jnp.float32),
    })

  def body(feats_h, src_h, dst_h, zrow_h, zcnt_h, ones_h, *outs, **sc):
    out_h = outs[0]
    core = lax.axis_index("c")
    sub = lax.axis_index("s")
    row0 = sub * RPS
    sidx, didx = sc["sidx"], sc["didx"]
    rows = [sc[f"rows{b}"] for b in range(NBUF)]
    gsem = [sc[f"gsem{b}"] for b in range(NBUF)]
    ssem = [sc[f"ssem{b}"] for b in range(NBUF)]
    acc = sc["acc"]

    pltpu.sync_copy(src_h.at[pl.ds((core * NS + sub) * STEPS, STEPS)], sidx)
    pltpu.sync_copy(dst_h.at[pl.ds(sub * STEPS, STEPS)], didx)

    pltpu.sync_copy(zrow_h, acc.at[pl.ds(row0, RPS)])
    if with_counts:
      pltpu.sync_copy(ones_h, sc["ones_v"])
      pltpu.sync_copy(zcnt_h, sc["cacc"].at[pl.ds(row0, RPS)])
    plsc.subcore_barrier()

    for b in range(NBUF):
      pltpu.async_copy(feats_h.at[sidx.at[b]], rows[b], gsem[b])

    def group(j, carry):
      k0 = j * NBUF
      handles = []
      for b in range(NBUF):
        k = k0 + b
        pltpu.make_async_copy(feats_h.at[sidx.at[k]], rows[b], gsem[b]).wait()
        handles.append(
            pltpu.async_copy(rows[b], acc.at[didx.at[k]], sem=ssem[b],
                             add=True))
        if with_counts:

          @pl.when(core == lax.rem(k, 2))
          def _():
            pltpu.async_copy(sc["ones_v"], sc["cacc"].at[didx.at[k]],
                             sem=sc[f"csem{b}"], add=True)

      for b in range(NBUF):
        k = k0 + b
        handles[b].wait()
        if with_counts:

          @pl.when(core == lax.rem(k, 2))
          def _():
            pltpu.make_async_copy(
                sc["ones_v"], sc["cacc"].at[didx.at[k]], sc[f"csem{b}"]).wait()

        @pl.when(k + NBUF < STEPS)
        def _():
          pltpu.async_copy(feats_h.at[sidx.at[k + NBUF]], rows[b], gsem[b])

      return carry

    lax.fori_loop(0, GROUPS, group, 0)
    plsc.subcore_barrier()

    pltpu.sync_copy(acc.at[pl.ds(row0, RPS)],
                    out_h.at[pl.ds(core * NPAD + row0, RPS)])
    if with_counts:

      @pl.when(core == 0)
      def _():
        pltpu.sync_copy(sc["cacc"].at[pl.ds(row0, RPS)],
                        outs[1].at[pl.ds(row0, RPS)])

      @pl.when(core == 1)
      def _():
        pltpu.sync_copy(sc["cacc"].at[pl.ds(row0, RPS)],
                        outs[2].at[pl.ds(row0, RPS)])

  fn = pl.kernel(
      body, out_type=out_type, mesh=mesh, scratch_types=scratch,
      compiler_params=pltpu.CompilerParams(use_tc_tiling_on_sc=False))
  return fn(feats2, srcx, dst4, zrow, zcnt, ones_pat)


def _dense_body(layer1, p_l, p_r, c0, c1, xr, wlt, bl, wrt, scale, beta, o):
  cnt = jnp.maximum(c0[:, 0:1] + c1[:, 0:1], 1.0)
  agg_l = p_l[...] / cnt
  agg_r = p_r[...] / cnt
  h = (jnp.dot(agg_l, wlt[0:HD, :], preferred_element_type=jnp.float32)
       + jnp.dot(agg_r, wlt[HD:D, :], preferred_element_type=jnp.float32)
       + bl[...]
       + jnp.dot(xr[...], wrt[...], preferred_element_type=jnp.float32))
  nrm = jnp.sqrt(jnp.sum(h * h, axis=1, keepdims=True))
  h = h / jnp.maximum(nrm, 1e-12)
  if layer1:
    h = h * scale[...] + beta[...]
    h = jnp.maximum(h, 0.0)
  o[...] = h


def _dense_call(layer1, P, C0, C1, xin, wlt, bl, wrt, scale, beta):
  R = 640
  NB = NPAD // R
  specs = [
      pl.BlockSpec((R, HD), lambda i: (i, 0)),
      pl.BlockSpec((R, HD), lambda i: (i + NB, 0)),
      pl.BlockSpec((R, L), lambda i: (i, 0)),
      pl.BlockSpec((R, L), lambda i: (i, 0)),
      pl.BlockSpec((R, D), lambda i: (i, 0)),
      pl.BlockSpec((D, D), lambda i: (0, 0)),
      pl.BlockSpec((1, D), lambda i: (0, 0)),
      pl.BlockSpec((D, D), lambda i: (0, 0)),
      pl.BlockSpec((1, D), lambda i: (0, 0)),
      pl.BlockSpec((1, D), lambda i: (0, 0)),
  ]
  return pl.pallas_call(
      functools.partial(_dense_body, layer1),
      grid=(NB,),
      in_specs=specs,
      out_specs=pl.BlockSpec((R, D), lambda i: (i, 0)),
      out_shape=jax.ShapeDtypeStruct((N, D), jnp.float32),
  )(P, P, C0, C1, xin, wlt, bl, wrt, scale, beta)


def kernel(x, edge_index, Wl1, bl1, Wr1, bn_gamma, bn_beta, Wl2, bl2, Wr2):
  src = edge_index[0].astype(jnp.int32)
  srcx = jnp.concatenate([2 * src, 2 * src + 1]).reshape(
      NC * NS * STEPS, CHUNK)
  dst4 = edge_index[1].astype(jnp.int32).reshape(NS * STEPS, CHUNK)
  zrow = jnp.zeros((RPS, HD), jnp.float32)
  zcnt = jnp.zeros((RPS, L), jnp.float32)
  ones_pat = jnp.zeros((CHUNK, L), jnp.float32).at[:, 0].set(1.0)

  P1, C0, C1 = _seg_sum_call(x.reshape(2 * N, HD), srcx, dst4, zrow, zcnt,
                             ones_pat, with_counts=True)
  scale1 = (bn_gamma / jnp.sqrt(1.0 + 1e-5)).reshape(1, D)
  h1 = _dense_call(True, P1, C0, C1, x, Wl1.T, bl1.reshape(1, D), Wr1.T,
                   scale1, bn_beta.reshape(1, D))

  (P2,) = _seg_sum_call(h1.reshape(2 * N, HD), srcx, dst4, zrow, zcnt,
                        ones_pat, with_counts=False)
  zb = jnp.zeros((1, D), jnp.float32)
  out = _dense_call(False, P2, C0, C1, h1, Wl2.T, bl2.reshape(1, D), Wr2.T,
                    zb, zb)
  return out


# speedup vs baseline: 2.8741x; 1.0569x over previous
import functools

import jax
import jax.numpy as jnp
from jax import lax
from jax.experimental import pallas as pl
from jax.experimental.pallas import tpu as pltpu
from jax.experimental.pallas import tpu_sc as plsc

N = 10000
E = 320000
D = 128
HD = D // 2
NC = 2
NS = 16
L = 16
EPS = E // NS
CHUNK = 80
STEPS = EPS // CHUNK
NBUF = 5
GROUPS = STEPS // NBUF
NPAD = 10240
RPS = NPAD // NS


def _seg_sum_call(feats2, srcx, dst4, zrow, zcnt, ones_pat, with_counts):
  mesh = plsc.VectorSubcoreMesh(
      core_axis_name="c", subcore_axis_name="s", num_cores=NC, num_subcores=NS)

  out_type = [jax.ShapeDtypeStruct((NC * NPAD, HD), jnp.float32)]
  scratch = {
      "sidx": pltpu.VMEM((STEPS, CHUNK), jnp.int32),
      "didx": pltpu.VMEM((STEPS, CHUNK), jnp.int32),
      "acc": pltpu.VMEM_SHARED((NPAD, HD), jnp.float32),
  }
  for b in range(NBUF):
    scratch[f"rows{b}"] = pltpu.VMEM((CHUNK, HD), jnp.float32)
    scratch[f"gsem{b}"] = pltpu.SemaphoreType.DMA
    scratch[f"ssem{b}"] = pltpu.SemaphoreType.DMA
    if with_counts:
      scratch[f"csem{b}"] = pltpu.SemaphoreType.DMA
  if with_counts:
    out_type.append(jax.ShapeDtypeStruct((NPAD, L), jnp.float32))
    out_type.append(jax.ShapeDtypeStruct((NPAD, L), jnp.float32))
    scratch.update({
        "ones_v": pltpu.VMEM((CHUNK, L), jnp.float32),
        "cacc": pltpu.VMEM_SHARED((NPAD, L), jnp.float32),
    })

  def body(feats_h, src_h, dst_h, zrow_h, zcnt_h, ones_h, *outs, **sc):
    out_h = outs[0]
    core = lax.axis_index("c")
    sub = lax.axis_index("s")
    row0 = sub * RPS
    sidx, didx = sc["sidx"], sc["didx"]
    rows = [sc[f"rows{b}"] for b in range(NBUF)]
    gsem = [sc[f"gsem{b}"] for b in range(NBUF)]
    ssem = [sc[f"ssem{b}"] for b in range(NBUF)]
    acc = sc["acc"]

    init = [
        pltpu.async_copy(src_h.at[pl.ds((core * NS + sub) * STEPS, STEPS)],
                         sidx, gsem[0]),
        pltpu.async_copy(dst_h.at[pl.ds(sub * STEPS, STEPS)], didx, gsem[1]),
        pltpu.async_copy(zrow_h, acc.at[pl.ds(row0, RPS)], gsem[2]),
    ]
    if with_counts:
      init.append(pltpu.async_copy(ones_h, sc["ones_v"], gsem[3]))
      init.append(
          pltpu.async_copy(zcnt_h, sc["cacc"].at[pl.ds(row0, RPS)], gsem[4]))
    for h in init:
      h.wait()
    plsc.subcore_barrier()

    for b in range(NBUF):
      pltpu.async_copy(feats_h.at[sidx.at[b]], rows[b], gsem[b])

    def group(j, carry):
      k0 = j * NBUF
      handles = []
      for b in range(NBUF):
        k = k0 + b
        pltpu.make_async_copy(feats_h.at[sidx.at[k]], rows[b], gsem[b]).wait()
        handles.append(
            pltpu.async_copy(rows[b], acc.at[didx.at[k]], sem=ssem[b],
                             add=True))
        if with_counts:

          @pl.when(core == lax.rem(k, 2))
          def _():
            pltpu.async_copy(sc["ones_v"], sc["cacc"].at[didx.at[k]],
                             sem=sc[f"csem{b}"], add=True)

      for b in range(NBUF):
        k = k0 + b
        handles[b].wait()
        if with_counts:

          @pl.when(core == lax.rem(k, 2))
          def _():
            pltpu.make_async_copy(
                sc["ones_v"], sc["cacc"].at[didx.at[k]], sc[f"csem{b}"]).wait()

        @pl.when(k + NBUF < STEPS)
        def _():
          pltpu.async_copy(feats_h.at[sidx.at[k + NBUF]], rows[b], gsem[b])

      return carry

    lax.fori_loop(0, GROUPS, group, 0)
    plsc.subcore_barrier()

    pltpu.sync_copy(acc.at[pl.ds(row0, RPS)],
                    out_h.at[pl.ds(core * NPAD + row0, RPS)])
    if with_counts:

      @pl.when(core == 0)
      def _():
        pltpu.sync_copy(sc["cacc"].at[pl.ds(row0, RPS)],
                        outs[1].at[pl.ds(row0, RPS)])

      @pl.when(core == 1)
      def _():
        pltpu.sync_copy(sc["cacc"].at[pl.ds(row0, RPS)],
                        outs[2].at[pl.ds(row0, RPS)])

  fn = pl.kernel(
      body, out_type=out_type, mesh=mesh, scratch_types=scratch,
      compiler_params=pltpu.CompilerParams(use_tc_tiling_on_sc=False))
  return fn(feats2, srcx, dst4, zrow, zcnt, ones_pat)


def _dense_body(layer1, p_l, p_r, c0, c1, xr, wlt, bl, wrt, scale, beta, o):
  cnt = jnp.maximum(c0[:, 0:1] + c1[:, 0:1], 1.0)
  agg_l = p_l[...] / cnt
  agg_r = p_r[...] / cnt
  h = (jnp.dot(agg_l, wlt[0:HD, :], preferred_element_type=jnp.float32)
       + jnp.dot(agg_r, wlt[HD:D, :], preferred_element_type=jnp.float32)
       + bl[...]
       + jnp.dot(xr[...], wrt[...], preferred_element_type=jnp.float32))
  nrm = jnp.sqrt(jnp.sum(h * h, axis=1, keepdims=True))
  h = h / jnp.maximum(nrm, 1e-12)
  if layer1:
    h = h * scale[...] + beta[...]
    h = jnp.maximum(h, 0.0)
  o[...] = h


def _dense_call(layer1, P, C0, C1, xin, wlt, bl, wrt, scale, beta):
  R = 2560
  NB = NPAD // R
  specs = [
      pl.BlockSpec((R, HD), lambda i: (i, 0)),
      pl.BlockSpec((R, HD), lambda i: (i + NB, 0)),
      pl.BlockSpec((R, L), lambda i: (i, 0)),
      pl.BlockSpec((R, L), lambda i: (i, 0)),
      pl.BlockSpec((R, D), lambda i: (i, 0)),
      pl.BlockSpec((D, D), lambda i: (0, 0)),
      pl.BlockSpec((1, D), lambda i: (0, 0)),
      pl.BlockSpec((D, D), lambda i: (0, 0)),
      pl.BlockSpec((1, D), lambda i: (0, 0)),
      pl.BlockSpec((1, D), lambda i: (0, 0)),
  ]
  return pl.pallas_call(
      functools.partial(_dense_body, layer1),
      grid=(NB,),
      in_specs=specs,
      out_specs=pl.BlockSpec((R, D), lambda i: (i, 0)),
      out_shape=jax.ShapeDtypeStruct((N, D), jnp.float32),
  )(P, P, C0, C1, xin, wlt, bl, wrt, scale, beta)


def kernel(x, edge_index, Wl1, bl1, Wr1, bn_gamma, bn_beta, Wl2, bl2, Wr2):
  src = edge_index[0].astype(jnp.int32)
  srcx = jnp.concatenate([2 * src, 2 * src + 1]).reshape(
      NC * NS * STEPS, CHUNK)
  dst4 = edge_index[1].astype(jnp.int32).reshape(NS * STEPS, CHUNK)
  zrow = jnp.zeros((RPS, HD), jnp.float32)
  zcnt = jnp.zeros((RPS, L), jnp.float32)
  ones_pat = jnp.zeros((CHUNK, L), jnp.float32).at[:, 0].set(1.0)

  P1, C0, C1 = _seg_sum_call(x.reshape(2 * N, HD), srcx, dst4, zrow, zcnt,
                             ones_pat, with_counts=True)
  scale1 = (bn_gamma / jnp.sqrt(1.0 + 1e-5)).reshape(1, D)
  h1 = _dense_call(True, P1, C0, C1, x, Wl1.T, bl1.reshape(1, D), Wr1.T,
                   scale1, bn_beta.reshape(1, D))

  (P2,) = _seg_sum_call(h1.reshape(2 * N, HD), srcx, dst4, zrow, zcnt,
                        ones_pat, with_counts=False)
  zb = jnp.zeros((1, D), jnp.float32)
  out = _dense_call(False, P2, C0, C1, h1, Wl2.T, bl2.reshape(1, D), Wr2.T,
                    zb, zb)
  return out
